# Initial kernel scaffold; baseline (speedup 1.0000x reference)
import jax
import jax.numpy as jnp
from jax.experimental import pallas as pl


def kernel(x, edge_index, W1, b1, W2, b2, Wl1, bl1, Wl2, bl2, Wl3, bl3):
    raise NotImplementedError("write your pallas kernel here")


# speedup vs baseline: 12.7678x; 12.7678x over previous
import functools

import jax
import jax.numpy as jnp
from jax import lax
from jax.experimental import pallas as pl
from jax.experimental.pallas import tpu as pltpu
from jax.experimental.pallas import tpu_sc as plsc

N = 10000
E = 320000
F = 128
NC = 2
NS = 16
NW = NC * NS
EP = E // NW
CH = 80
NCH = EP // CH
RCH = 80
NRCH = N // RCH
KPT = -(-NRCH // NS)


def _mesh():
    return plsc.VectorSubcoreMesh(core_axis_name="c", subcore_axis_name="s",
                                  num_cores=NC, num_subcores=NS)


@functools.cache
def _make_deg_kernel():
    @functools.partial(
        pl.kernel,
        out_type=jax.ShapeDtypeStruct((NW, 1, N), jnp.float32),
        mesh=_mesh(),
        scratch_types=[
            pltpu.VMEM((CH,), jnp.int32),
            pltpu.VMEM((N,), jnp.float32),
        ],
        compiler_params=pltpu.CompilerParams(needs_layout_passes=False),
    )
    def deg_kernel(dst_hbm, out_hbm, didx, degv):
        c = lax.axis_index("c")
        s = lax.axis_index("s")
        wid = s * NC + c

        def zero(i, _):
            degv[pl.ds(i * 16, 16)] = jnp.zeros((16,), jnp.float32)
            return 0

        lax.fori_loop(0, N // 16, zero, 0)

        base0 = wid * EP
        ones = jnp.ones((16,), jnp.float32)

        def body(i, _):
            base = pl.multiple_of(base0 + i * CH, 8)
            pltpu.sync_copy(dst_hbm.at[pl.ds(base, CH)], didx)
            for j in range(CH // 16):
                dv = didx[pl.ds(j * 16, 16)]
                plsc.addupdate_scatter(degv, [dv], ones)
            return 0

        lax.fori_loop(0, NCH, body, 0)
        pltpu.sync_copy(degv, out_hbm.at[wid, 0])

    return deg_kernel


@functools.cache
def _make_agg_kernel():
    @functools.partial(
        pl.kernel,
        out_type=jax.ShapeDtypeStruct((NC, N, F), jnp.float32),
        mesh=_mesh(),
        scratch_types=[
            pltpu.VMEM((CH,), jnp.int32),
            pltpu.VMEM((CH,), jnp.int32),
            pltpu.VMEM((CH, F), jnp.float32),
            pltpu.VMEM((RCH, F), jnp.float32),
            pltpu.VMEM_SHARED((N, F), jnp.float32),
            pltpu.SemaphoreType.DMA,
        ],
    )
    def agg_kernel(hs_hbm, src_hbm, dst_hbm, out_hbm, sidx, didx, rows, stage,
                   acc, sem):
        c = lax.axis_index("c")
        s = lax.axis_index("s")
        wid = s * NC + c

        def zero(t, _):
            r = t // (F // 16)
            j = t % (F // 16)
            stage[r, pl.ds(j * 16, 16)] = jnp.zeros((16,), jnp.float32)
            return 0

        lax.fori_loop(0, RCH * (F // 16), zero, 0)
        for k in range(KPT):
            ri = s * KPT + k

            @pl.when(ri < NRCH)
            def _():
                pltpu.sync_copy(stage, acc.at[pl.ds(ri * RCH, RCH)])
        plsc.subcore_barrier()

        base0 = wid * EP

        def body(i, _):
            base = pl.multiple_of(base0 + i * CH, 8)
            pltpu.sync_copy(src_hbm.at[pl.ds(base, CH)], sidx)
            pltpu.sync_copy(dst_hbm.at[pl.ds(base, CH)], didx)
            pltpu.async_copy(hs_hbm.at[sidx], rows, sem).wait()
            pltpu.sync_copy(rows, acc.at[didx], add=True)
            return 0

        lax.fori_loop(0, NCH, body, 0)
        plsc.subcore_barrier()

        for k in range(KPT):
            ri = s * KPT + k

            @pl.when(ri < NRCH)
            def _():
                r0 = pl.multiple_of(ri * RCH, 8)
                pltpu.sync_copy(acc.at[pl.ds(r0, RCH)], stage)
                pltpu.sync_copy(stage, out_hbm.at[c, pl.ds(r0, RCH)])

    return agg_kernel


R = 1000
GRID = N // R


def _dinv_block(degt):
    deg = jnp.sum(degt, axis=1, keepdims=True) + 1.0
    return lax.rsqrt(deg)


def _tc_a_body(x_ref, w_ref, degt_ref, o_ref):
    dinv = _dinv_block(degt_ref[...])
    h = jnp.dot(x_ref[...], w_ref[...], preferred_element_type=jnp.float32)
    o_ref[...] = h * dinv


def _tc_b_body(p_ref, h1s_ref, degt_ref, b1_ref, w2_ref, o_ref):
    dinv = _dinv_block(degt_ref[...])
    p = p_ref[...]
    out1 = jnp.maximum(dinv * (p[0] + p[1] + h1s_ref[...]) + b1_ref[...], 0.0)
    h2 = jnp.dot(out1, w2_ref[...], preferred_element_type=jnp.float32)
    o_ref[...] = h2 * dinv


def _tc_c_body(p_ref, h2s_ref, degt_ref, b2_ref, wl1_ref, bl1_ref, wl2_ref,
               bl2_ref, wl3_ref, bl3_ref, o_ref):
    dinv = _dinv_block(degt_ref[...])
    p = p_ref[...]
    out2 = jnp.maximum(dinv * (p[0] + p[1] + h2s_ref[...]) + b2_ref[...], 0.0)
    g = jnp.maximum(
        jnp.dot(out2, wl1_ref[...], preferred_element_type=jnp.float32)
        + bl1_ref[...], 0.0)
    g = jnp.maximum(
        jnp.dot(g, wl2_ref[...], preferred_element_type=jnp.float32)
        + bl2_ref[...], 0.0)
    o_ref[...] = (jnp.dot(g, wl3_ref[...], preferred_element_type=jnp.float32)
                  + bl3_ref[...])


def _row_spec(f):
    return pl.BlockSpec((R, f), lambda i: (i, 0))


def _full_spec(shape):
    nd = len(shape)
    return pl.BlockSpec(shape, lambda i, _n=nd: (0,) * _n)


_DEGT_SPEC = pl.BlockSpec((R, NW), lambda i: (i, 0))
_P_SPEC = pl.BlockSpec((NC, R, F), lambda i: (0, i, 0))


def _tc_a(x, W1, degt):
    return pl.pallas_call(
        _tc_a_body,
        grid=(GRID,),
        in_specs=[_row_spec(128), _full_spec((128, 128)), _DEGT_SPEC],
        out_specs=_row_spec(128),
        out_shape=jax.ShapeDtypeStruct((N, 128), jnp.float32),
    )(x, W1, degt)


def _tc_b(p1, h1s, degt, b1r, W2p):
    return pl.pallas_call(
        _tc_b_body,
        grid=(GRID,),
        in_specs=[
            _P_SPEC,
            _row_spec(128),
            _DEGT_SPEC,
            _full_spec((1, 128)),
            _full_spec((128, 128)),
        ],
        out_specs=_row_spec(128),
        out_shape=jax.ShapeDtypeStruct((N, 128), jnp.float32),
    )(p1, h1s, degt, b1r, W2p)


def _tc_c(p2, h2s, degt, b2r, Wl1p, bl1r, Wl2, bl2r, Wl3, bl3r):
    return pl.pallas_call(
        _tc_c_body,
        grid=(GRID,),
        in_specs=[
            _P_SPEC,
            _row_spec(128),
            _DEGT_SPEC,
            _full_spec((1, 128)),
            _full_spec((128, 512)),
            _full_spec((1, 512)),
            _full_spec((512, 512)),
            _full_spec((1, 512)),
            _full_spec((512, 2)),
            _full_spec((1, 2)),
        ],
        out_specs=_row_spec(2),
        out_shape=jax.ShapeDtypeStruct((N, 2), jnp.float32),
    )(p2, h2s, degt, b2r, Wl1p, bl1r, Wl2, bl2r, Wl3, bl3r)


def kernel(x, edge_index, W1, b1, W2, b2, Wl1, bl1, Wl2, bl2, Wl3, bl3):
    src = edge_index[0]
    dst = edge_index[1]

    W2p = jnp.pad(W2, ((0, 0), (0, 128 - 12)))
    b2p = jnp.pad(b2, (0, 128 - 12))
    Wl1p = jnp.pad(Wl1, ((0, 128 - 12), (0, 0)))

    degp = _make_deg_kernel()(dst)
    degt = jnp.transpose(degp.reshape(NW, N))
    h1s = _tc_a(x, W1, degt)
    p1 = _make_agg_kernel()(h1s, src, dst)
    h2s = _tc_b(p1, h1s, degt, b1.reshape(1, 128), W2p)
    p2 = _make_agg_kernel()(h2s, src, dst)
    out = _tc_c(p2, h2s, degt, b2p.reshape(1, 128), Wl1p, bl1.reshape(1, 512),
                Wl2, bl2.reshape(1, 512), Wl3, bl3.reshape(1, 2))
    return out


# speedup vs baseline: 29.2456x; 2.2906x over previous
import functools

import jax
import jax.numpy as jnp
from jax import lax
from jax.experimental import pallas as pl
from jax.experimental.pallas import tpu as pltpu
from jax.experimental.pallas import tpu_sc as plsc

N = 10000
E = 320000
F = 128
NC = 2
NS = 16
NW = NC * NS
EP = E // NW
CH = 80
NCH = EP // CH
RCH = 80
NRCH = N // RCH
KPT = -(-NRCH // NS)


def _mesh():
    return plsc.VectorSubcoreMesh(core_axis_name="c", subcore_axis_name="s",
                                  num_cores=NC, num_subcores=NS)


@functools.cache
def _make_deg_kernel():
    @functools.partial(
        pl.kernel,
        out_type=jax.ShapeDtypeStruct((NW, 1, N), jnp.float32),
        mesh=_mesh(),
        scratch_types=[
            pltpu.VMEM((NCH, CH), jnp.int32),
            pltpu.VMEM((N,), jnp.float32),
        ],
        compiler_params=pltpu.CompilerParams(needs_layout_passes=False),
    )
    def deg_kernel(dst_hbm, out_hbm, didx, degv):
        c = lax.axis_index("c")
        s = lax.axis_index("s")
        wid = s * NC + c

        pltpu.sync_copy(dst_hbm.at[wid], didx)

        def zero(i, _):
            degv[pl.ds(i * 16, 16)] = jnp.zeros((16,), jnp.float32)
            return 0

        lax.fori_loop(0, N // 16, zero, 0)

        ones = jnp.ones((16,), jnp.float32)

        def body(i, _):
            for j in range(CH // 16):
                dv = didx[i, pl.ds(j * 16, 16)]
                plsc.addupdate_scatter(degv, [dv], ones)
            return 0

        lax.fori_loop(0, NCH, body, 0)
        pltpu.sync_copy(degv, out_hbm.at[wid, 0])

    return deg_kernel


GB = 2


@functools.cache
def _make_agg_kernel():
    @functools.partial(
        pl.kernel,
        out_type=jax.ShapeDtypeStruct((NC, N, F), jnp.float32),
        mesh=_mesh(),
        scratch_types=[
            pltpu.VMEM((EP,), jnp.int32),
            pltpu.VMEM((NCH, CH), jnp.int32),
            pltpu.VMEM((GB, CH, F), jnp.float32),
            pltpu.VMEM_SHARED((N, F), jnp.float32),
            pltpu.SemaphoreType.DMA((GB,)),
        ],
    )
    def agg_kernel(hs_hbm, src_hbm, dst_hbm, out_hbm, sidx, didx, rows,
                   acc, sem):
        c = lax.axis_index("c")
        s = lax.axis_index("s")
        wid = s * NC + c

        pltpu.sync_copy(src_hbm.at[wid, 0], sidx)
        pltpu.sync_copy(dst_hbm.at[wid], didx)

        stage = rows.at[0]

        def zero(t, _):
            r = t // (F // 16)
            j = t % (F // 16)
            stage[r, pl.ds(j * 16, 16)] = jnp.zeros((16,), jnp.float32)
            return 0

        lax.fori_loop(0, RCH * (F // 16), zero, 0)
        for k in range(KPT):
            ri = s * KPT + k

            @pl.when(ri < NRCH)
            def _():
                pltpu.sync_copy(stage, acc.at[pl.ds(ri * RCH, RCH)])
        plsc.subcore_barrier()

        def src_slice(i):
            return sidx.at[pl.ds(pl.multiple_of(i * CH, 8), CH)]

        def gather(i, b):
            pltpu.async_copy(hs_hbm.at[src_slice(i)], rows.at[b], sem.at[b])

        def gather_wait(i, b):
            pltpu.make_async_copy(hs_hbm.at[src_slice(i)], rows.at[b],
                                  sem.at[b]).wait()

        for b in range(GB):
            gather(b, b)

        M = NCH - NCH % GB

        @pl.loop(0, M, step=GB)
        def _(i0):
            for b in range(GB):
                i = i0 + b
                gather_wait(i, b)
                pltpu.sync_copy(rows.at[b], acc.at[didx.at[i]], add=True)

                @pl.when(i + GB < NCH)
                def _():
                    gather(i + GB, b)

        for i in range(M, NCH):
            b = i % GB
            gather_wait(i, b)
            pltpu.sync_copy(rows.at[b], acc.at[didx.at[i]], add=True)

        plsc.subcore_barrier()

        out_stage = rows.at[0]
        for k in range(KPT):
            ri = s * KPT + k

            @pl.when(ri < NRCH)
            def _():
                r0 = pl.multiple_of(ri * RCH, 8)
                pltpu.sync_copy(acc.at[pl.ds(r0, RCH)], out_stage)
                pltpu.sync_copy(out_stage, out_hbm.at[c, pl.ds(r0, RCH)])

    return agg_kernel


R = 1000
GRID = N // R


def _dinv_block(degt):
    deg = jnp.sum(degt, axis=1, keepdims=True) + 1.0
    return lax.rsqrt(deg)


def _tc_a_body(x_ref, w_ref, degt_ref, o_ref):
    dinv = _dinv_block(degt_ref[...])
    h = jnp.dot(x_ref[...], w_ref[...], preferred_element_type=jnp.float32)
    o_ref[...] = h * dinv


def _tc_b_body(p_ref, h1s_ref, degt_ref, b1_ref, w2_ref, o_ref):
    dinv = _dinv_block(degt_ref[...])
    p = p_ref[...]
    out1 = jnp.maximum(dinv * (p[0] + p[1] + h1s_ref[...]) + b1_ref[...], 0.0)
    h2 = jnp.dot(out1, w2_ref[...], preferred_element_type=jnp.float32)
    o_ref[...] = h2 * dinv


def _tc_c_body(p_ref, h2s_ref, degt_ref, b2_ref, wl1_ref, bl1_ref, wl2_ref,
               bl2_ref, wl3_ref, bl3_ref, o_ref):
    dinv = _dinv_block(degt_ref[...])
    p = p_ref[...]
    out2 = jnp.maximum(dinv * (p[0] + p[1] + h2s_ref[...]) + b2_ref[...], 0.0)
    g = jnp.maximum(
        jnp.dot(out2, wl1_ref[...], preferred_element_type=jnp.float32)
        + bl1_ref[...], 0.0)
    g = jnp.maximum(
        jnp.dot(g, wl2_ref[...], preferred_element_type=jnp.float32)
        + bl2_ref[...], 0.0)
    o_ref[...] = (jnp.dot(g, wl3_ref[...], preferred_element_type=jnp.float32)
                  + bl3_ref[...])


def _row_spec(f):
    return pl.BlockSpec((R, f), lambda i: (i, 0))


def _full_spec(shape):
    nd = len(shape)
    return pl.BlockSpec(shape, lambda i, _n=nd: (0,) * _n)


_DEGT_SPEC = pl.BlockSpec((R, NW), lambda i: (i, 0))
_P_SPEC = pl.BlockSpec((NC, R, F), lambda i: (0, i, 0))


def _tc_a(x, W1, degt):
    return pl.pallas_call(
        _tc_a_body,
        grid=(GRID,),
        in_specs=[_row_spec(128), _full_spec((128, 128)), _DEGT_SPEC],
        out_specs=_row_spec(128),
        out_shape=jax.ShapeDtypeStruct((N, 128), jnp.float32),
    )(x, W1, degt)


def _tc_b(p1, h1s, degt, b1r, W2p):
    return pl.pallas_call(
        _tc_b_body,
        grid=(GRID,),
        in_specs=[
            _P_SPEC,
            _row_spec(128),
            _DEGT_SPEC,
            _full_spec((1, 128)),
            _full_spec((128, 128)),
        ],
        out_specs=_row_spec(128),
        out_shape=jax.ShapeDtypeStruct((N, 128), jnp.float32),
    )(p1, h1s, degt, b1r, W2p)


def _tc_c(p2, h2s, degt, b2r, Wl1p, bl1r, Wl2, bl2r, Wl3, bl3r):
    return pl.pallas_call(
        _tc_c_body,
        grid=(GRID,),
        in_specs=[
            _P_SPEC,
            _row_spec(128),
            _DEGT_SPEC,
            _full_spec((1, 128)),
            _full_spec((128, 512)),
            _full_spec((1, 512)),
            _full_spec((512, 512)),
            _full_spec((1, 512)),
            _full_spec((512, 2)),
            _full_spec((1, 2)),
        ],
        out_specs=_row_spec(2),
        out_shape=jax.ShapeDtypeStruct((N, 2), jnp.float32),
    )(p2, h2s, degt, b2r, Wl1p, bl1r, Wl2, bl2r, Wl3, bl3r)


def kernel(x, edge_index, W1, b1, W2, b2, Wl1, bl1, Wl2, bl2, Wl3, bl3):
    src = edge_index[0].reshape(NW, 1, EP)
    dst = edge_index[1].reshape(NW, NCH, CH)

    W2p = jnp.pad(W2, ((0, 0), (0, 128 - 12)))
    b2p = jnp.pad(b2, (0, 128 - 12))
    Wl1p = jnp.pad(Wl1, ((0, 128 - 12), (0, 0)))

    degp = _make_deg_kernel()(dst)
    degt = jnp.transpose(degp.reshape(NW, N))
    h1s = _tc_a(x, W1, degt)
    p1 = _make_agg_kernel()(h1s, src, dst)
    h2s = _tc_b(p1, h1s, degt, b1.reshape(1, 128), W2p)
    p2 = _make_agg_kernel()(h2s, src, dst)
    out = _tc_c(p2, h2s, degt, b2p.reshape(1, 128), Wl1p, bl1.reshape(1, 512),
                Wl2, bl2.reshape(1, 512), Wl3, bl3.reshape(1, 2))
    return out


# speedup vs baseline: 34.3219x; 1.1736x over previous
import functools

import jax
import jax.numpy as jnp
from jax import lax
from jax.experimental import pallas as pl
from jax.experimental.pallas import tpu as pltpu
from jax.experimental.pallas import tpu_sc as plsc

N = 10000
E = 320000
F = 128
NC = 2
NS = 16
NW = NC * NS
EP = E // NW
CH = 80
NCH = EP // CH
RCH = 80
NRCH = N // RCH
KPT = -(-NRCH // NS)


def _mesh():
    return plsc.VectorSubcoreMesh(core_axis_name="c", subcore_axis_name="s",
                                  num_cores=NC, num_subcores=NS)


@functools.cache
def _make_deg_kernel():
    @functools.partial(
        pl.kernel,
        out_type=jax.ShapeDtypeStruct((NW, 1, N), jnp.float32),
        mesh=_mesh(),
        scratch_types=[
            pltpu.VMEM((NCH, CH), jnp.int32),
            pltpu.VMEM((N,), jnp.float32),
        ],
        compiler_params=pltpu.CompilerParams(needs_layout_passes=False),
    )
    def deg_kernel(dst_hbm, out_hbm, didx, degv):
        c = lax.axis_index("c")
        s = lax.axis_index("s")
        wid = s * NC + c

        pltpu.sync_copy(dst_hbm.at[wid], didx)

        def zero(i, _):
            degv[pl.ds(i * 16, 16)] = jnp.zeros((16,), jnp.float32)
            return 0

        lax.fori_loop(0, N // 16, zero, 0)

        ones = jnp.ones((16,), jnp.float32)

        def body(i, _):
            for j in range(CH // 16):
                dv = didx[i, pl.ds(j * 16, 16)]
                plsc.addupdate_scatter(degv, [dv], ones)
            return 0

        lax.fori_loop(0, NCH, body, 0)
        pltpu.sync_copy(degv, out_hbm.at[wid, 0])

    return deg_kernel


GB = 2


@functools.cache
def _make_agg_kernel(F, tc_tiling=True):
    @functools.partial(
        pl.kernel,
        out_type=jax.ShapeDtypeStruct((NC, N, F), jnp.float32),
        mesh=_mesh(),
        scratch_types=[
            pltpu.VMEM((EP,), jnp.int32),
            pltpu.VMEM((NCH, CH), jnp.int32),
            pltpu.VMEM((GB, CH, F), jnp.float32),
            pltpu.VMEM_SHARED((N, F), jnp.float32),
            pltpu.SemaphoreType.DMA((GB,)),
        ],
        compiler_params=(None if tc_tiling else
                         pltpu.CompilerParams(use_tc_tiling_on_sc=False)),
    )
    def agg_kernel(hs_hbm, src_hbm, dst_hbm, out_hbm, sidx, didx, rows,
                   acc, sem):
        c = lax.axis_index("c")
        s = lax.axis_index("s")
        wid = s * NC + c

        pltpu.sync_copy(src_hbm.at[wid, 0], sidx)
        pltpu.sync_copy(dst_hbm.at[wid], didx)

        stage = rows.at[0]

        def zero(t, _):
            r = t // (F // 16)
            j = t % (F // 16)
            stage[r, pl.ds(j * 16, 16)] = jnp.zeros((16,), jnp.float32)
            return 0

        lax.fori_loop(0, RCH * (F // 16), zero, 0)
        for k in range(KPT):
            ri = s * KPT + k

            @pl.when(ri < NRCH)
            def _():
                pltpu.sync_copy(stage, acc.at[pl.ds(ri * RCH, RCH)])
        plsc.subcore_barrier()

        def src_slice(i):
            return sidx.at[pl.ds(pl.multiple_of(i * CH, 8), CH)]

        def gather(i, b):
            pltpu.async_copy(hs_hbm.at[src_slice(i)], rows.at[b], sem.at[b])

        def gather_wait(i, b):
            pltpu.make_async_copy(hs_hbm.at[src_slice(i)], rows.at[b],
                                  sem.at[b]).wait()

        for b in range(GB):
            gather(b, b)

        M = NCH - NCH % GB

        @pl.loop(0, M, step=GB)
        def _(i0):
            for b in range(GB):
                i = i0 + b
                gather_wait(i, b)
                pltpu.sync_copy(rows.at[b], acc.at[didx.at[i]], add=True)

                @pl.when(i + GB < NCH)
                def _():
                    gather(i + GB, b)

        for i in range(M, NCH):
            b = i % GB
            gather_wait(i, b)
            pltpu.sync_copy(rows.at[b], acc.at[didx.at[i]], add=True)

        plsc.subcore_barrier()

        out_stage = rows.at[0]
        for k in range(KPT):
            ri = s * KPT + k

            @pl.when(ri < NRCH)
            def _():
                r0 = pl.multiple_of(ri * RCH, 8)
                pltpu.sync_copy(acc.at[pl.ds(r0, RCH)], out_stage)
                pltpu.sync_copy(out_stage, out_hbm.at[c, pl.ds(r0, RCH)])

    return agg_kernel


R = 1000
GRID = N // R


def _dinv_block(degt):
    deg = jnp.sum(degt, axis=1, keepdims=True) + 1.0
    return lax.rsqrt(deg)


def _tc_a_body(x_ref, w_ref, degt_ref, o_ref):
    dinv = _dinv_block(degt_ref[...])
    h = jnp.dot(x_ref[...], w_ref[...], preferred_element_type=jnp.float32)
    o_ref[...] = h * dinv


def _tc_b_body(p_ref, h1s_ref, degt_ref, b1_ref, w2_ref, o_ref):
    dinv = _dinv_block(degt_ref[...])
    p = p_ref[...]
    out1 = jnp.maximum(dinv * (p[0] + p[1] + h1s_ref[...]) + b1_ref[...], 0.0)
    h2 = jnp.dot(out1, w2_ref[...], preferred_element_type=jnp.float32)
    o_ref[...] = h2 * dinv


def _tc_c_body(p_ref, h2s_ref, degt_ref, b2_ref, wl1_ref, bl1_ref, wl2_ref,
               bl2_ref, wl3_ref, bl3_ref, o_ref):
    dinv = _dinv_block(degt_ref[...])
    p = p_ref[...]
    out2 = jnp.maximum(dinv * (p[0] + p[1] + h2s_ref[...]) + b2_ref[...], 0.0)
    g = jnp.maximum(
        jnp.dot(out2, wl1_ref[...], preferred_element_type=jnp.float32)
        + bl1_ref[...], 0.0)
    g = jnp.maximum(
        jnp.dot(g, wl2_ref[...], preferred_element_type=jnp.float32)
        + bl2_ref[...], 0.0)
    o_ref[...] = (jnp.dot(g, wl3_ref[...], preferred_element_type=jnp.float32)
                  + bl3_ref[...])


def _row_spec(f):
    return pl.BlockSpec((R, f), lambda i: (i, 0))


def _full_spec(shape):
    nd = len(shape)
    return pl.BlockSpec(shape, lambda i, _n=nd: (0,) * _n)


_DEGT_SPEC = pl.BlockSpec((R, NW), lambda i: (i, 0))
_P_SPEC = pl.BlockSpec((NC, R, F), lambda i: (0, i, 0))


def _tc_a(x, W1, degt):
    return pl.pallas_call(
        _tc_a_body,
        grid=(GRID,),
        in_specs=[_row_spec(128), _full_spec((128, 128)), _DEGT_SPEC],
        out_specs=_row_spec(128),
        out_shape=jax.ShapeDtypeStruct((N, 128), jnp.float32),
    )(x, W1, degt)


def _tc_b(p1, h1s, degt, b1r, W2p):
    return pl.pallas_call(
        _tc_b_body,
        grid=(GRID,),
        in_specs=[
            _P_SPEC,
            _row_spec(128),
            _DEGT_SPEC,
            _full_spec((1, 128)),
            _full_spec((128, 16)),
        ],
        out_specs=_row_spec(16),
        out_shape=jax.ShapeDtypeStruct((N, 16), jnp.float32),
    )(p1, h1s, degt, b1r, W2p)


def _tc_c(p2, h2s, degt, b2r, Wl1p, bl1r, Wl2, bl2r, Wl3, bl3r):
    return pl.pallas_call(
        _tc_c_body,
        grid=(GRID,),
        in_specs=[
            pl.BlockSpec((NC, R, 16), lambda i: (0, i, 0)),
            _row_spec(16),
            _DEGT_SPEC,
            _full_spec((1, 16)),
            _full_spec((16, 512)),
            _full_spec((1, 512)),
            _full_spec((512, 512)),
            _full_spec((1, 512)),
            _full_spec((512, 2)),
            _full_spec((1, 2)),
        ],
        out_specs=_row_spec(2),
        out_shape=jax.ShapeDtypeStruct((N, 2), jnp.float32),
    )(p2, h2s, degt, b2r, Wl1p, bl1r, Wl2, bl2r, Wl3, bl3r)


def kernel(x, edge_index, W1, b1, W2, b2, Wl1, bl1, Wl2, bl2, Wl3, bl3):
    src = edge_index[0].reshape(NW, 1, EP)
    dst = edge_index[1].reshape(NW, NCH, CH)

    W2p = jnp.pad(W2, ((0, 0), (0, 4)))
    b2p = jnp.pad(b2, (0, 4))
    Wl1p = jnp.pad(Wl1, ((0, 4), (0, 0)))

    degp = _make_deg_kernel()(dst)
    degt = jnp.transpose(degp.reshape(NW, N))
    h1s = _tc_a(x, W1, degt)
    p1 = _make_agg_kernel(128)(h1s, src, dst)
    h2s = _tc_b(p1, h1s, degt, b1.reshape(1, 128), W2p)
    p2 = _make_agg_kernel(16, tc_tiling=False)(h2s, src, dst)
    out = _tc_c(p2, h2s, degt, b2p.reshape(1, 16), Wl1p, bl1.reshape(1, 512),
                Wl2, bl2.reshape(1, 512), Wl3, bl3.reshape(1, 2))
    return out


# speedup vs baseline: 39.0743x; 1.1385x over previous
import functools

import jax
import jax.numpy as jnp
from jax import lax
from jax.experimental import pallas as pl
from jax.experimental.pallas import tpu as pltpu
from jax.experimental.pallas import tpu_sc as plsc

N = 10000
E = 320000
F = 128
NC = 2
NS = 16
NW = NC * NS
EP = E // NW
CH = 80
NCH = EP // CH
RCH = 80
NRCH = N // RCH
KPT = -(-NRCH // NS)


def _mesh():
    return plsc.VectorSubcoreMesh(core_axis_name="c", subcore_axis_name="s",
                                  num_cores=NC, num_subcores=NS)


@functools.cache
def _make_deg_kernel():
    @functools.partial(
        pl.kernel,
        out_type=jax.ShapeDtypeStruct((NW, 1, N), jnp.float32),
        mesh=_mesh(),
        scratch_types=[
            pltpu.VMEM((NCH, CH), jnp.int32),
            pltpu.VMEM((N,), jnp.float32),
        ],
        compiler_params=pltpu.CompilerParams(needs_layout_passes=False),
    )
    def deg_kernel(dst_hbm, out_hbm, didx, degv):
        c = lax.axis_index("c")
        s = lax.axis_index("s")
        wid = s * NC + c

        pltpu.sync_copy(dst_hbm.at[wid], didx)

        def zero(i, _):
            degv[pl.ds(i * 16, 16)] = jnp.zeros((16,), jnp.float32)
            return 0

        lax.fori_loop(0, N // 16, zero, 0)

        ones = jnp.ones((16,), jnp.float32)

        def body(i, _):
            for j in range(CH // 16):
                dv = didx[i, pl.ds(j * 16, 16)]
                plsc.addupdate_scatter(degv, [dv], ones)
            return 0

        lax.fori_loop(0, NCH, body, 0)
        pltpu.sync_copy(degv, out_hbm.at[wid, 0])

    return deg_kernel


@functools.cache
def _make_agg_kernel(F, tc_tiling=True):
    GB = 2 if F == 128 else 8

    @functools.partial(
        pl.kernel,
        out_type=jax.ShapeDtypeStruct((NC, N, F), jnp.float32),
        mesh=_mesh(),
        scratch_types=[
            pltpu.VMEM((EP,), jnp.int32),
            pltpu.VMEM((NCH, CH), jnp.int32),
            pltpu.VMEM((GB, CH, F), jnp.float32),
            pltpu.VMEM_SHARED((N, F), jnp.float32),
            pltpu.SemaphoreType.DMA((GB,)),
            pltpu.SemaphoreType.DMA((GB,)),
        ],
        compiler_params=(None if tc_tiling else
                         pltpu.CompilerParams(use_tc_tiling_on_sc=False)),
    )
    def agg_kernel(hs_hbm, src_hbm, dst_hbm, out_hbm, sidx, didx, rows,
                   acc, sem, ssem):
        c = lax.axis_index("c")
        s = lax.axis_index("s")
        wid = s * NC + c

        pltpu.sync_copy(src_hbm.at[wid, 0], sidx)
        pltpu.sync_copy(dst_hbm.at[wid], didx)

        stage = rows.at[0]

        def zero(t, _):
            r = t // (F // 16)
            j = t % (F // 16)
            stage[r, pl.ds(j * 16, 16)] = jnp.zeros((16,), jnp.float32)
            return 0

        lax.fori_loop(0, RCH * (F // 16), zero, 0)
        for k in range(KPT):
            ri = s * KPT + k

            @pl.when(ri < NRCH)
            def _():
                pltpu.sync_copy(stage, acc.at[pl.ds(ri * RCH, RCH)])
        plsc.subcore_barrier()

        def src_slice(i):
            return sidx.at[pl.ds(pl.multiple_of(i * CH, 8), CH)]

        def gather(i, b):
            pltpu.async_copy(hs_hbm.at[src_slice(i)], rows.at[b], sem.at[b])

        def gather_wait(i, b):
            pltpu.make_async_copy(hs_hbm.at[src_slice(i)], rows.at[b],
                                  sem.at[b]).wait()

        def scatter(i, b):
            pltpu.async_copy(rows.at[b], acc.at[didx.at[i]], ssem.at[b],
                             add=True)

        def scatter_wait(i, b):
            pltpu.make_async_copy(rows.at[b], acc.at[didx.at[i]],
                                  ssem.at[b]).wait()

        for b in range(GB):
            gather(b, b)

        M = NCH - NCH % GB

        @pl.loop(0, M, step=GB)
        def _(i0):
            for b in range(GB):
                i = i0 + b
                gather_wait(i, b)
                scatter(i, b)

                @pl.when(i + GB < NCH)
                def _():
                    scatter_wait(i, b)
                    gather(i + GB, b)

        for i in range(M, NCH):
            b = i % GB
            gather_wait(i, b)
            scatter(i, b)

        for i in range(NCH - GB, NCH):
            scatter_wait(i, i % GB)

        plsc.subcore_barrier()

        out_stage = rows.at[0]
        for k in range(KPT):
            ri = s * KPT + k

            @pl.when(ri < NRCH)
            def _():
                r0 = pl.multiple_of(ri * RCH, 8)
                pltpu.sync_copy(acc.at[pl.ds(r0, RCH)], out_stage)
                pltpu.sync_copy(out_stage, out_hbm.at[c, pl.ds(r0, RCH)])

    return agg_kernel


R = 1000
GRID = N // R


def _dinv_block(degt):
    deg = jnp.sum(degt, axis=1, keepdims=True) + 1.0
    return lax.rsqrt(deg)


def _tc_a_body(x_ref, w_ref, degt_ref, o_ref):
    dinv = _dinv_block(degt_ref[...])
    h = jnp.dot(x_ref[...], w_ref[...], preferred_element_type=jnp.float32)
    o_ref[...] = h * dinv


def _tc_b_body(p_ref, h1s_ref, degt_ref, b1_ref, w2_ref, o_ref):
    dinv = _dinv_block(degt_ref[...])
    p = p_ref[...]
    out1 = jnp.maximum(dinv * (p[0] + p[1] + h1s_ref[...]) + b1_ref[...], 0.0)
    h2 = jnp.dot(out1, w2_ref[...], preferred_element_type=jnp.float32)
    o_ref[...] = h2 * dinv


def _tc_c_body(p_ref, h2s_ref, degt_ref, b2_ref, wl1_ref, bl1_ref, wl2_ref,
               bl2_ref, wl3_ref, bl3_ref, o_ref):
    dinv = _dinv_block(degt_ref[...])
    p = p_ref[...]
    out2 = jnp.maximum(dinv * (p[0] + p[1] + h2s_ref[...]) + b2_ref[...], 0.0)
    g = jnp.maximum(
        jnp.dot(out2, wl1_ref[...], preferred_element_type=jnp.float32)
        + bl1_ref[...], 0.0)
    g = jnp.maximum(
        jnp.dot(g, wl2_ref[...], preferred_element_type=jnp.float32)
        + bl2_ref[...], 0.0)
    o_ref[...] = (jnp.dot(g, wl3_ref[...], preferred_element_type=jnp.float32)
                  + bl3_ref[...])


def _row_spec(f):
    return pl.BlockSpec((R, f), lambda i: (i, 0))


def _full_spec(shape):
    nd = len(shape)
    return pl.BlockSpec(shape, lambda i, _n=nd: (0,) * _n)


_DEGT_SPEC = pl.BlockSpec((R, NW), lambda i: (i, 0))
_P_SPEC = pl.BlockSpec((NC, R, F), lambda i: (0, i, 0))


def _tc_a(x, W1, degt):
    return pl.pallas_call(
        _tc_a_body,
        grid=(GRID,),
        in_specs=[_row_spec(128), _full_spec((128, 128)), _DEGT_SPEC],
        out_specs=_row_spec(128),
        out_shape=jax.ShapeDtypeStruct((N, 128), jnp.float32),
    )(x, W1, degt)


def _tc_b(p1, h1s, degt, b1r, W2p):
    return pl.pallas_call(
        _tc_b_body,
        grid=(GRID,),
        in_specs=[
            _P_SPEC,
            _row_spec(128),
            _DEGT_SPEC,
            _full_spec((1, 128)),
            _full_spec((128, 16)),
        ],
        out_specs=_row_spec(16),
        out_shape=jax.ShapeDtypeStruct((N, 16), jnp.float32),
    )(p1, h1s, degt, b1r, W2p)


def _tc_c(p2, h2s, degt, b2r, Wl1p, bl1r, Wl2, bl2r, Wl3, bl3r):
    return pl.pallas_call(
        _tc_c_body,
        grid=(GRID,),
        in_specs=[
            pl.BlockSpec((NC, R, 16), lambda i: (0, i, 0)),
            _row_spec(16),
            _DEGT_SPEC,
            _full_spec((1, 16)),
            _full_spec((16, 512)),
            _full_spec((1, 512)),
            _full_spec((512, 512)),
            _full_spec((1, 512)),
            _full_spec((512, 2)),
            _full_spec((1, 2)),
        ],
        out_specs=_row_spec(2),
        out_shape=jax.ShapeDtypeStruct((N, 2), jnp.float32),
    )(p2, h2s, degt, b2r, Wl1p, bl1r, Wl2, bl2r, Wl3, bl3r)


def kernel(x, edge_index, W1, b1, W2, b2, Wl1, bl1, Wl2, bl2, Wl3, bl3):
    src = edge_index[0].reshape(NW, 1, EP)
    dst = edge_index[1].reshape(NW, NCH, CH)

    W2p = jnp.pad(W2, ((0, 0), (0, 4)))
    b2p = jnp.pad(b2, (0, 4))
    Wl1p = jnp.pad(Wl1, ((0, 4), (0, 0)))

    degp = _make_deg_kernel()(dst)
    degt = jnp.transpose(degp.reshape(NW, N))
    h1s = _tc_a(x, W1, degt)
    p1 = _make_agg_kernel(128)(h1s, src, dst)
    h2s = _tc_b(p1, h1s, degt, b1.reshape(1, 128), W2p)
    p2 = _make_agg_kernel(16, tc_tiling=False)(h2s, src, dst)
    out = _tc_c(p2, h2s, degt, b2p.reshape(1, 16), Wl1p, bl1.reshape(1, 512),
                Wl2, bl2.reshape(1, 512), Wl3, bl3.reshape(1, 2))
    return out


# speedup vs baseline: 39.2211x; 1.0038x over previous
import functools

import jax
import jax.numpy as jnp
from jax import lax
from jax.experimental import pallas as pl
from jax.experimental.pallas import tpu as pltpu
from jax.experimental.pallas import tpu_sc as plsc

N = 10000
E = 320000
F = 128
NC = 2
NS = 16
NW = NC * NS
EP = E // NW
CH = 80
NCH = EP // CH
RCH = 80
NRCH = N // RCH
KPT = -(-NRCH // NS)


def _mesh():
    return plsc.VectorSubcoreMesh(core_axis_name="c", subcore_axis_name="s",
                                  num_cores=NC, num_subcores=NS)


@functools.cache
def _make_deg_kernel():
    @functools.partial(
        pl.kernel,
        out_type=jax.ShapeDtypeStruct((NW, 1, N), jnp.float32),
        mesh=_mesh(),
        scratch_types=[
            pltpu.VMEM((EP,), jnp.int32),
            pltpu.VMEM((N,), jnp.float32),
        ],
        compiler_params=pltpu.CompilerParams(needs_layout_passes=False),
    )
    def deg_kernel(dst_hbm, out_hbm, didx, degv):
        c = lax.axis_index("c")
        s = lax.axis_index("s")
        wid = s * NC + c

        base0 = pl.multiple_of(wid * EP, 8)
        pltpu.sync_copy(dst_hbm.at[pl.ds(base0, EP)], didx)

        def zero(i, _):
            degv[pl.ds(i * 16, 16)] = jnp.zeros((16,), jnp.float32)
            return 0

        lax.fori_loop(0, N // 16, zero, 0)

        ones = jnp.ones((16,), jnp.float32)

        def body(i, _):
            for j in range(CH // 16):
                dv = didx[pl.ds(i * CH + j * 16, 16)]
                plsc.addupdate_scatter(degv, [dv], ones)
            return 0

        lax.fori_loop(0, NCH, body, 0)
        pltpu.sync_copy(degv, out_hbm.at[wid, 0])

    return deg_kernel


@functools.cache
def _make_agg_kernel(F, tc_tiling=True):
    GB = 2 if F == 128 else 8

    @functools.partial(
        pl.kernel,
        out_type=jax.ShapeDtypeStruct((NC, N, F), jnp.float32),
        mesh=_mesh(),
        scratch_types=[
            pltpu.VMEM((EP,), jnp.int32),
            pltpu.VMEM((NCH, CH) if tc_tiling else (EP,), jnp.int32),
            pltpu.VMEM((GB, CH, F), jnp.float32),
            pltpu.VMEM_SHARED((N, F), jnp.float32),
            pltpu.SemaphoreType.DMA((GB,)),
            pltpu.SemaphoreType.DMA((GB,)),
        ],
        compiler_params=(None if tc_tiling else
                         pltpu.CompilerParams(use_tc_tiling_on_sc=False)),
    )
    def agg_kernel(hs_hbm, src_hbm, dst_hbm, out_hbm, sidx, didx, rows,
                   acc, sem, ssem):
        c = lax.axis_index("c")
        s = lax.axis_index("s")
        wid = s * NC + c

        ebase = pl.multiple_of(wid * EP, 8)
        pltpu.sync_copy(src_hbm.at[pl.ds(ebase, EP)], sidx)
        if tc_tiling:
            pltpu.sync_copy(dst_hbm.at[wid], didx)
        else:
            pltpu.sync_copy(dst_hbm.at[pl.ds(ebase, EP)], didx)

        stage = rows.at[0]

        def zero(t, _):
            r = t // (F // 16)
            j = t % (F // 16)
            stage[r, pl.ds(j * 16, 16)] = jnp.zeros((16,), jnp.float32)
            return 0

        lax.fori_loop(0, RCH * (F // 16), zero, 0)
        for k in range(KPT):
            ri = s * KPT + k

            @pl.when(ri < NRCH)
            def _():
                pltpu.sync_copy(stage, acc.at[pl.ds(ri * RCH, RCH)])
        plsc.subcore_barrier()

        def src_slice(i):
            return sidx.at[pl.ds(pl.multiple_of(i * CH, 8), CH)]

        def dst_slice(i):
            if tc_tiling:
                return didx.at[i]
            return didx.at[pl.ds(pl.multiple_of(i * CH, 8), CH)]

        def gather(i, b):
            pltpu.async_copy(hs_hbm.at[src_slice(i)], rows.at[b], sem.at[b])

        def gather_wait(i, b):
            pltpu.make_async_copy(hs_hbm.at[src_slice(i)], rows.at[b],
                                  sem.at[b]).wait()

        def scatter(i, b):
            pltpu.async_copy(rows.at[b], acc.at[dst_slice(i)], ssem.at[b],
                             add=True)

        def scatter_wait(i, b):
            pltpu.make_async_copy(rows.at[b], acc.at[dst_slice(i)],
                                  ssem.at[b]).wait()

        for b in range(GB):
            gather(b, b)

        M = NCH - NCH % GB

        @pl.loop(0, M, step=GB)
        def _(i0):
            for b in range(GB):
                i = i0 + b
                gather_wait(i, b)
                scatter(i, b)

                @pl.when(i + GB < NCH)
                def _():
                    scatter_wait(i, b)
                    gather(i + GB, b)

        for i in range(M, NCH):
            b = i % GB
            gather_wait(i, b)
            scatter(i, b)

        for i in range(NCH - GB, NCH):
            scatter_wait(i, i % GB)

        plsc.subcore_barrier()

        out_stage = rows.at[0]
        for k in range(KPT):
            ri = s * KPT + k

            @pl.when(ri < NRCH)
            def _():
                r0 = pl.multiple_of(ri * RCH, 8)
                pltpu.sync_copy(acc.at[pl.ds(r0, RCH)], out_stage)
                pltpu.sync_copy(out_stage, out_hbm.at[c, pl.ds(r0, RCH)])

    return agg_kernel


R = 1000
GRID = N // R


def _dinv_block(degt):
    deg = jnp.sum(degt, axis=1, keepdims=True) + 1.0
    return lax.rsqrt(deg)


def _tc_a_body(x_ref, w_ref, degt_ref, o_ref):
    dinv = _dinv_block(degt_ref[...])
    h = jnp.dot(x_ref[...], w_ref[...], preferred_element_type=jnp.float32)
    o_ref[...] = h * dinv


def _tc_b_body(p_ref, h1s_ref, degt_ref, b1_ref, w2_ref, o_ref):
    dinv = _dinv_block(degt_ref[...])
    p = p_ref[...]
    out1 = jnp.maximum(dinv * (p[0] + p[1] + h1s_ref[...]) + b1_ref[...], 0.0)
    h2 = jnp.dot(out1, w2_ref[...], preferred_element_type=jnp.float32)
    o_ref[...] = h2 * dinv


def _tc_c_body(p_ref, h2s_ref, degt_ref, b2_ref, wl1_ref, bl1_ref, wl2_ref,
               bl2_ref, wl3_ref, bl3_ref, o_ref):
    dinv = _dinv_block(degt_ref[...])
    p = p_ref[...]
    out2 = jnp.maximum(dinv * (p[0] + p[1] + h2s_ref[...]) + b2_ref[...], 0.0)
    g = jnp.maximum(
        jnp.dot(out2, wl1_ref[...], preferred_element_type=jnp.float32)
        + bl1_ref[...], 0.0)
    g = jnp.maximum(
        jnp.dot(g, wl2_ref[...], preferred_element_type=jnp.float32)
        + bl2_ref[...], 0.0)
    o_ref[...] = (jnp.dot(g, wl3_ref[...], preferred_element_type=jnp.float32)
                  + bl3_ref[...])


def _row_spec(f):
    return pl.BlockSpec((R, f), lambda i: (i, 0))


def _full_spec(shape):
    nd = len(shape)
    return pl.BlockSpec(shape, lambda i, _n=nd: (0,) * _n)


_DEGT_SPEC = pl.BlockSpec((R, NW), lambda i: (i, 0))
_P_SPEC = pl.BlockSpec((NC, R, F), lambda i: (0, i, 0))


def _tc_a(x, W1, degt):
    return pl.pallas_call(
        _tc_a_body,
        grid=(GRID,),
        in_specs=[_row_spec(128), _full_spec((128, 128)), _DEGT_SPEC],
        out_specs=_row_spec(128),
        out_shape=jax.ShapeDtypeStruct((N, 128), jnp.float32),
    )(x, W1, degt)


def _tc_b(p1, h1s, degt, b1r, W2p):
    return pl.pallas_call(
        _tc_b_body,
        grid=(GRID,),
        in_specs=[
            _P_SPEC,
            _row_spec(128),
            _DEGT_SPEC,
            _full_spec((1, 128)),
            _full_spec((128, 16)),
        ],
        out_specs=_row_spec(16),
        out_shape=jax.ShapeDtypeStruct((N, 16), jnp.float32),
    )(p1, h1s, degt, b1r, W2p)


def _tc_c(p2, h2s, degt, b2r, Wl1p, bl1r, Wl2, bl2r, Wl3, bl3r):
    return pl.pallas_call(
        _tc_c_body,
        grid=(GRID,),
        in_specs=[
            pl.BlockSpec((NC, R, 16), lambda i: (0, i, 0)),
            _row_spec(16),
            _DEGT_SPEC,
            _full_spec((1, 16)),
            _full_spec((16, 512)),
            _full_spec((1, 512)),
            _full_spec((512, 512)),
            _full_spec((1, 512)),
            _full_spec((512, 2)),
            _full_spec((1, 2)),
        ],
        out_specs=_row_spec(2),
        out_shape=jax.ShapeDtypeStruct((N, 2), jnp.float32),
    )(p2, h2s, degt, b2r, Wl1p, bl1r, Wl2, bl2r, Wl3, bl3r)


def kernel(x, edge_index, W1, b1, W2, b2, Wl1, bl1, Wl2, bl2, Wl3, bl3):
    src = edge_index[0]
    dst = edge_index[1]
    dstR = dst.reshape(NW, NCH, CH)

    W2p = jnp.pad(W2, ((0, 0), (0, 4)))
    b2p = jnp.pad(b2, (0, 4))
    Wl1p = jnp.pad(Wl1, ((0, 4), (0, 0)))

    degp = _make_deg_kernel()(dst)
    degt = jnp.transpose(degp.reshape(NW, N))
    h1s = _tc_a(x, W1, degt)
    p1 = _make_agg_kernel(128)(h1s, src, dstR)
    h2s = _tc_b(p1, h1s, degt, b1.reshape(1, 128), W2p)
    p2 = _make_agg_kernel(16, tc_tiling=False)(h2s, src, dst)
    out = _tc_c(p2, h2s, degt, b2p.reshape(1, 16), Wl1p, bl1.reshape(1, 512),
                Wl2, bl2.reshape(1, 512), Wl3, bl3.reshape(1, 2))
    return out
